# baseline (speedup 1.0000x reference)
import functools

import jax
import jax.numpy as jnp
from jax import lax
from jax.experimental import pallas as pl
from jax.experimental.pallas import tpu as pltpu
from jax.experimental.pallas import tpu_sc as plsc

_CHUNK = 64


@functools.lru_cache(maxsize=None)
def _build(V, D, BATCH, SEQ):
    info = plsc.get_sparse_core_info()
    NC, NS, L = info.num_cores, info.num_subcores, info.num_lanes
    NW = NC * NS
    B = BATCH * SEQ
    assert SEQ == NW * _CHUNK and D % L == 0
    rows_per_w = BATCH * _CHUNK
    scale = float(D) ** 0.5
    mesh = plsc.VectorSubcoreMesh(core_axis_name="c", subcore_axis_name="s")

    @functools.partial(
        pl.kernel,
        mesh=mesh,
        out_type=jax.ShapeDtypeStruct((BATCH, SEQ, D), jnp.float32),
        scratch_types=[
            pltpu.VMEM((BATCH, _CHUNK), jnp.int32),
            pltpu.VMEM((rows_per_w, D), jnp.float32),
            pltpu.VMEM((_CHUNK, D), jnp.float32),
            pltpu.SemaphoreType.DMA,
            pltpu.SemaphoreType.DMA,
            pltpu.SemaphoreType.DMA,
            pltpu.SemaphoreType.DMA,
        ],
    )
    def emb_kernel(idx_hbm, table_hbm, pe_hbm, out_hbm, idx_v, rows_v, pe_v,
                   sem_i, sem_g, sem_pe, sem_w):
        c = lax.axis_index("c")
        s = lax.axis_index("s")
        t = s * NC + c

        idx_cps = [
            pltpu.async_copy(idx_hbm.at[b, pl.ds(t * _CHUNK, _CHUNK)], idx_v.at[b], sem_i)
            for b in range(BATCH)
        ]
        gathers = []
        for b in range(BATCH):
            idx_cps[b].wait()
            gathers.append(
                pltpu.async_copy(
                    table_hbm.at[idx_v.at[b]],
                    rows_v.at[pl.ds(b * _CHUNK, _CHUNK)],
                    sem_g,
                )
            )
        pe_cp = pltpu.async_copy(pe_hbm.at[pl.ds(t * _CHUNK, _CHUNK)], pe_v, sem_pe)
        pe_cp.wait()

        writes = []
        for b in range(BATCH):
            gathers[b].wait()
            splits = (
                [(0, _CHUNK)] if b < BATCH - 1
                else [(0, _CHUNK // 2), (_CHUNK // 2, _CHUNK)]
            )
            for lo, hi in splits:

                @plsc.parallel_loop(lo, hi, step=1, unroll=1)
                def body(i):
                    for j in range(D // L):
                        sl = pl.ds(j * L, L)
                        rows_v[b * _CHUNK + i, sl] = (
                            rows_v[b * _CHUNK + i, sl] * scale + pe_v[i, sl]
                        )

                writes.append(
                    pltpu.async_copy(
                        rows_v.at[pl.ds(b * _CHUNK + lo, hi - lo)],
                        out_hbm.at[b, pl.ds(t * _CHUNK + lo, hi - lo)],
                        sem_w,
                    )
                )
        for wcp in writes:
            wcp.wait()

    return emb_kernel


def kernel(src_ids, W, pe):
    BATCH, SEQ = src_ids.shape
    V, D = W.shape
    return _build(V, D, BATCH, SEQ)(src_ids.astype(jnp.int32), W, pe)


# speedup vs baseline: 1.0153x; 1.0153x over previous
import functools

import jax
import jax.numpy as jnp
from jax import lax
from jax.experimental import pallas as pl
from jax.experimental.pallas import tpu as pltpu
from jax.experimental.pallas import tpu_sc as plsc

_CHUNK = 64


@functools.lru_cache(maxsize=None)
def _build(V, D, BATCH, SEQ):
    info = plsc.get_sparse_core_info()
    NC, NS, L = info.num_cores, info.num_subcores, info.num_lanes
    NW = NC * NS
    B = BATCH * SEQ
    assert SEQ == NW * _CHUNK and D % L == 0
    rows_per_w = BATCH * _CHUNK
    scale = float(D) ** 0.5
    mesh = plsc.VectorSubcoreMesh(core_axis_name="c", subcore_axis_name="s")

    @functools.partial(
        pl.kernel,
        mesh=mesh,
        out_type=jax.ShapeDtypeStruct((BATCH, SEQ, D), jnp.float32),
        scratch_types=[
            pltpu.VMEM((BATCH, _CHUNK), jnp.int32),
            pltpu.VMEM((rows_per_w, D), jnp.float32),
            pltpu.VMEM((_CHUNK, D), jnp.float32),
            pltpu.SemaphoreType.DMA,
            pltpu.SemaphoreType.DMA,
            pltpu.SemaphoreType.DMA,
            pltpu.SemaphoreType.DMA,
        ],
    )
    def emb_kernel(idx_hbm, table_hbm, pe_hbm, out_hbm, idx_v, rows_v, pe_v,
                   sem_i, sem_g, sem_pe, sem_w):
        c = lax.axis_index("c")
        s = lax.axis_index("s")
        t = s * NC + c

        pe_cp = pltpu.async_copy(pe_hbm.at[pl.ds(t * _CHUNK, _CHUNK)], pe_v, sem_pe)
        idx_cps = [
            pltpu.async_copy(idx_hbm.at[b, pl.ds(t * _CHUNK, _CHUNK)], idx_v.at[b], sem_i)
            for b in range(BATCH)
        ]
        for cp in idx_cps:
            cp.wait()
        gathers = [
            pltpu.async_copy(
                table_hbm.at[idx_v.at[b]],
                rows_v.at[pl.ds(b * _CHUNK, _CHUNK)],
                sem_g,
            )
            for b in range(BATCH)
        ]
        pe_cp.wait()

        writes = []
        for b in range(BATCH):
            gathers[b].wait()
            splits = (
                [(0, _CHUNK)] if b < BATCH - 1
                else [(0, _CHUNK // 2), (_CHUNK // 2, _CHUNK)]
            )
            for lo, hi in splits:

                @plsc.parallel_loop(lo, hi, step=1, unroll=1)
                def body(i):
                    for j in range(D // L):
                        sl = pl.ds(j * L, L)
                        rows_v[b * _CHUNK + i, sl] = (
                            rows_v[b * _CHUNK + i, sl] * scale + pe_v[i, sl]
                        )

                writes.append(
                    pltpu.async_copy(
                        rows_v.at[pl.ds(b * _CHUNK + lo, hi - lo)],
                        out_hbm.at[b, pl.ds(t * _CHUNK + lo, hi - lo)],
                        sem_w,
                    )
                )
        for wcp in writes:
            wcp.wait()

    return emb_kernel


def kernel(src_ids, W, pe):
    BATCH, SEQ = src_ids.shape
    V, D = W.shape
    return _build(V, D, BATCH, SEQ)(src_ids.astype(jnp.int32), W, pe)
